# baseline (speedup 1.0000x reference)
import functools

import jax
import jax.numpy as jnp
from jax import lax
from jax.experimental import pallas as pl
from jax.experimental.pallas import tpu as pltpu
from jax.experimental.pallas import tpu_sc as plsc

IN_DIM = 1000000
OUT_DIM = 64
BATCH = 16384
HIST = 50

NC = 2
NS = 16
NW = NC * NS

NT0 = BATCH // 128
T0W = NT0 // NW
NUNIT = HIST * T0W
NBUF = 4
OBUF = 2


def _body(w2_hbm, idx_hbm, y_hbm, idxbuf, halfbuf, selbuf, gbuf, obuf,
          isem, gsem, osem):
    wid = lax.axis_index("s") * NC + lax.axis_index("c")
    t0_base = wid * T0W
    iota16 = jnp.arange(16, dtype=jnp.int32)

    def unit_hw(u):
        h = lax.div(u, T0W)
        t0 = t0_base + lax.rem(u, T0W)
        return h, t0

    def fire_idx(u, slot):
        h, t0 = unit_hw(u)
        pltpu.async_copy(idx_hbm.at[h, t0], idxbuf.at[slot], isem.at[slot])

    def wait_idx(slot):
        pltpu.make_async_copy(
            idx_hbm.at[0, 0], idxbuf.at[slot], isem.at[slot]).wait()

    def vecprep(slot):
        for s8 in range(8):
            sl = pl.ds(s8 * 16, 16)
            v = idxbuf[slot, sl]
            halfbuf[slot, sl] = lax.shift_right_logical(v, 1)
            selbuf[slot, sl] = lax.shift_left(jnp.bitwise_and(v, 1), 6)

    def fire_gather(slot):
        pltpu.async_copy(w2_hbm.at[halfbuf.at[slot]], gbuf.at[slot],
                         gsem.at[slot])

    def wait_gather(slot):
        pltpu.make_async_copy(
            w2_hbm.at[pl.ds(0, 128)], gbuf.at[slot], gsem.at[slot]).wait()

    def transpose(slot, oslot):
        @pl.loop(0, OUT_DIM)
        def _j(j):
            jd = lax.div(j, 8)
            jr = lax.rem(j, 8)
            for s8 in range(8):
                sl = pl.ds(s8 * 16, 16)
                rowv = iota16 + (s8 * 16)
                colv = selbuf[slot, sl] + j
                obuf[oslot, jd, jr, sl] = plsc.load_gather(
                    gbuf.at[slot], [rowv, colv])

    def fire_out(u, oslot):
        h, t0 = unit_hw(u)
        pltpu.async_copy(obuf.at[oslot],
                         y_hbm.at[pl.ds(h * 8, 8), t0], osem.at[oslot])

    def wait_out(oslot):
        pltpu.make_async_copy(
            y_hbm.at[pl.ds(0, 8), 0], obuf.at[oslot], osem.at[oslot]).wait()

    for un in range(NBUF):
        fire_idx(un, un)
    for un in range(NBUF - 1):
        wait_idx(un)
        vecprep(un)
        fire_gather(un)

    @pl.loop(0, NUNIT, step=NBUF)
    def _outer(u0):
        for b in range(NBUF):
            u = u0 + b
            slot3 = (b + NBUF - 1) % NBUF
            oslot = b % OBUF

            @pl.when(u + NBUF < NUNIT)
            def _idx():
                fire_idx(u + NBUF, b)

            @pl.when(u + (NBUF - 1) < NUNIT)
            def _refill():
                wait_idx(slot3)
                vecprep(slot3)
                fire_gather(slot3)

            wait_gather(b)

            @pl.when(u >= OBUF)
            def _wout():
                wait_out(oslot)

            transpose(b, oslot)
            fire_out(u, oslot)

    for oslot in range(OBUF):
        wait_out(oslot)


@jax.jit
def _embedding_gather(idx3, w2):
    mesh = plsc.VectorSubcoreMesh(
        core_axis_name="c", subcore_axis_name="s",
        num_cores=NC, num_subcores=NS,
    )
    run = functools.partial(
        pl.kernel,
        mesh=mesh,
        out_type=jax.ShapeDtypeStruct((HIST * 8, NT0, 8, 128), jnp.float32),
        scratch_types=[
            pltpu.VMEM((NBUF, 128), jnp.int32),
            pltpu.VMEM((NBUF, 128), jnp.int32),
            pltpu.VMEM((NBUF, 128), jnp.int32),
            pltpu.VMEM((NBUF, 128, 128), jnp.float32),
            pltpu.VMEM((OBUF, 8, 8, 128), jnp.float32),
            pltpu.SemaphoreType.DMA((NBUF,)),
            pltpu.SemaphoreType.DMA((NBUF,)),
            pltpu.SemaphoreType.DMA((OBUF,)),
        ],
        compiler_params=pltpu.CompilerParams(
            use_tc_tiling_on_sc=False, needs_layout_passes=False),
    )(_body)
    return run(w2, idx3)


def kernel(tensor, weights):
    w2 = weights.reshape(IN_DIM // 2, 2 * OUT_DIM)
    idx3 = jnp.transpose(tensor).reshape(HIST, NT0, 128).astype(jnp.int32)
    y = _embedding_gather(idx3, w2)
    out = (y.reshape(HIST, 8, NT0, 8, 128)
            .transpose(2, 4, 0, 1, 3)
            .reshape(BATCH, HIST, OUT_DIM))
    return out


# speedup vs baseline: 1.3720x; 1.3720x over previous
import functools

import jax
import jax.numpy as jnp
from jax import lax
from jax.experimental import pallas as pl
from jax.experimental.pallas import tpu as pltpu
from jax.experimental.pallas import tpu_sc as plsc

IN_DIM = 1000000
OUT_DIM = 64
BATCH = 16384
HIST = 50

NC = 2
NS = 16
NW = NC * NS

NT0 = BATCH // 128
T0W = NT0 // NW
NUNIT = HIST * T0W
NBUF = 4
OBUF = 2


def _body(w2_hbm, idx_hbm, y_hbm, idxbuf, halfbuf, selbuf, gbuf, obuf,
          isem, gsem, osem):
    wid = lax.axis_index("s") * NC + lax.axis_index("c")
    t0_base = wid * T0W
    iota16 = jnp.arange(16, dtype=jnp.int32)

    def unit_hw(u):
        h = lax.div(u, T0W)
        t0 = t0_base + lax.rem(u, T0W)
        return h, t0

    def fire_idx(u, slot):
        h, t0 = unit_hw(u)
        pltpu.async_copy(idx_hbm.at[h, t0], idxbuf.at[slot], isem.at[slot])

    def wait_idx(slot):
        pltpu.make_async_copy(
            idx_hbm.at[0, 0], idxbuf.at[slot], isem.at[slot]).wait()

    def vecprep(slot):
        for s8 in range(8):
            sl = pl.ds(s8 * 16, 16)
            v = idxbuf[slot, sl]
            halfbuf[slot, sl] = lax.shift_right_logical(v, 1)
            selbuf[slot, sl] = lax.shift_left(jnp.bitwise_and(v, 1), 6)

    def fire_gather(slot):
        pltpu.async_copy(w2_hbm.at[halfbuf.at[slot]], gbuf.at[slot],
                         gsem.at[slot])

    def wait_gather(slot):
        pltpu.make_async_copy(
            w2_hbm.at[pl.ds(0, 128)], gbuf.at[slot], gsem.at[slot]).wait()

    rowvs = [iota16 + s8 * 16 for s8 in range(8)]

    def transpose(slot, oslot):
        sels = [selbuf[slot, pl.ds(s8 * 16, 16)] for s8 in range(8)]

        @pl.loop(0, 8)
        def _jd(jd):
            j0 = jd * 8
            for jr in range(8):
                for s8 in range(8):
                    obuf[oslot, jd, jr, pl.ds(s8 * 16, 16)] = plsc.load_gather(
                        gbuf.at[slot], [rowvs[s8], sels[s8] + (j0 + jr)])

    def fire_out(u, oslot):
        h, t0 = unit_hw(u)
        pltpu.async_copy(obuf.at[oslot],
                         y_hbm.at[pl.ds(h * 8, 8), t0], osem.at[oslot])

    def wait_out(oslot):
        pltpu.make_async_copy(
            y_hbm.at[pl.ds(0, 8), 0], obuf.at[oslot], osem.at[oslot]).wait()

    for un in range(NBUF):
        fire_idx(un, un)
    for un in range(NBUF - 1):
        wait_idx(un)
        vecprep(un)
        fire_gather(un)

    @pl.loop(0, NUNIT, step=NBUF)
    def _outer(u0):
        for b in range(NBUF):
            u = u0 + b
            slot3 = (b + NBUF - 1) % NBUF
            oslot = b % OBUF

            @pl.when(u + NBUF < NUNIT)
            def _idx():
                fire_idx(u + NBUF, b)

            @pl.when(u + (NBUF - 1) < NUNIT)
            def _refill():
                wait_idx(slot3)
                vecprep(slot3)
                fire_gather(slot3)

            wait_gather(b)

            @pl.when(u >= OBUF)
            def _wout():
                wait_out(oslot)

            transpose(b, oslot)
            fire_out(u, oslot)

    for oslot in range(OBUF):
        wait_out(oslot)


@jax.jit
def _embedding_gather(idx3, w2):
    mesh = plsc.VectorSubcoreMesh(
        core_axis_name="c", subcore_axis_name="s",
        num_cores=NC, num_subcores=NS,
    )
    run = functools.partial(
        pl.kernel,
        mesh=mesh,
        out_type=jax.ShapeDtypeStruct((HIST * 8, NT0, 8, 128), jnp.float32),
        scratch_types=[
            pltpu.VMEM((NBUF, 128), jnp.int32),
            pltpu.VMEM((NBUF, 128), jnp.int32),
            pltpu.VMEM((NBUF, 128), jnp.int32),
            pltpu.VMEM((NBUF, 128, 128), jnp.float32),
            pltpu.VMEM((OBUF, 8, 8, 128), jnp.float32),
            pltpu.SemaphoreType.DMA((NBUF,)),
            pltpu.SemaphoreType.DMA((NBUF,)),
            pltpu.SemaphoreType.DMA((OBUF,)),
        ],
        compiler_params=pltpu.CompilerParams(
            use_tc_tiling_on_sc=False, needs_layout_passes=False),
    )(_body)
    return run(w2, idx3)


def kernel(tensor, weights):
    w2 = weights.reshape(IN_DIM // 2, 2 * OUT_DIM)
    idx3 = jnp.transpose(tensor).reshape(HIST, NT0, 128).astype(jnp.int32)
    y = _embedding_gather(idx3, w2)
    out = (y.reshape(HIST, 8, NT0, 8, 128)
            .transpose(2, 4, 0, 1, 3)
            .reshape(BATCH, HIST, OUT_DIM))
    return out


# speedup vs baseline: 2.4824x; 1.8093x over previous
import functools

import jax
import jax.numpy as jnp
from jax import lax
from jax.experimental import pallas as pl
from jax.experimental.pallas import tpu as pltpu
from jax.experimental.pallas import tpu_sc as plsc

IN_DIM = 1000000
OUT_DIM = 64
BATCH = 16384
HIST = 50

NC = 2
NS = 16
NW = NC * NS

NT0 = BATCH // 128
T0W = NT0 // NW
NUNIT = HIST * T0W
NBUF = 4
OBUF = 2


def _body(w2_hbm, idx_hbm, y_hbm, idxbuf, halfbuf, selbuf, gbuf, obuf,
          isem, gsem, osem):
    wid = lax.axis_index("s") * NC + lax.axis_index("c")
    t0_base = wid * T0W
    iota16 = jnp.arange(16, dtype=jnp.int32)

    def unit_hw(u):
        h = lax.div(u, T0W)
        t0 = t0_base + lax.rem(u, T0W)
        return h, t0

    def fire_idx(u, slot):
        h, t0 = unit_hw(u)
        pltpu.async_copy(idx_hbm.at[h, t0], idxbuf.at[slot], isem.at[slot])

    def wait_idx(slot):
        pltpu.make_async_copy(
            idx_hbm.at[0, 0], idxbuf.at[slot], isem.at[slot]).wait()

    def vecprep(slot):
        for s8 in range(8):
            sl = pl.ds(s8 * 16, 16)
            v = idxbuf[slot, sl]
            halfbuf[slot, sl] = lax.shift_right_logical(v, 1)
            selbuf[slot, sl] = lax.shift_left(jnp.bitwise_and(v, 1), 6)

    def fire_gather(slot):
        pltpu.async_copy(w2_hbm.at[halfbuf.at[slot]], gbuf.at[slot],
                         gsem.at[slot])

    def wait_gather(slot):
        pltpu.make_async_copy(
            w2_hbm.at[pl.ds(0, 128)], gbuf.at[slot], gsem.at[slot]).wait()

    rowvs = [iota16 + s8 * 16 for s8 in range(8)]

    def transpose(slot, oslot):
        sels = [selbuf[slot, pl.ds(s8 * 16, 16)] for s8 in range(8)]

        @pl.loop(0, 4)
        def _jb(jb):
            j0 = jb * 16

            @pl.loop(0, 16, unroll=4)
            def _k(k):
                perm = jnp.bitwise_and(iota16 + k, 15)
                jv = j0 + perm
                jdv = lax.shift_right_logical(jv, 3)
                jrv = jnp.bitwise_and(jv, 7)
                for s8 in range(8):
                    val = plsc.load_gather(
                        gbuf.at[slot], [rowvs[s8], sels[s8] + jv])
                    plsc.store_scatter(
                        obuf.at[oslot], [jdv, jrv, rowvs[s8]], val)

    def fire_out(u, oslot):
        h, t0 = unit_hw(u)
        pltpu.async_copy(obuf.at[oslot],
                         y_hbm.at[pl.ds(h * 8, 8), t0], osem.at[oslot])

    def wait_out(oslot):
        pltpu.make_async_copy(
            y_hbm.at[pl.ds(0, 8), 0], obuf.at[oslot], osem.at[oslot]).wait()

    for un in range(NBUF):
        fire_idx(un, un)
    for un in range(NBUF - 1):
        wait_idx(un)
        vecprep(un)
        fire_gather(un)

    @pl.loop(0, NUNIT, step=NBUF)
    def _outer(u0):
        for b in range(NBUF):
            u = u0 + b
            slot3 = (b + NBUF - 1) % NBUF
            oslot = b % OBUF

            @pl.when(u + NBUF < NUNIT)
            def _idx():
                fire_idx(u + NBUF, b)

            @pl.when(u + (NBUF - 1) < NUNIT)
            def _refill():
                wait_idx(slot3)
                vecprep(slot3)
                fire_gather(slot3)

            wait_gather(b)

            @pl.when(u >= OBUF)
            def _wout():
                wait_out(oslot)

            transpose(b, oslot)
            fire_out(u, oslot)

    for oslot in range(OBUF):
        wait_out(oslot)


@jax.jit
def _embedding_gather(idx3, w2):
    mesh = plsc.VectorSubcoreMesh(
        core_axis_name="c", subcore_axis_name="s",
        num_cores=NC, num_subcores=NS,
    )
    run = functools.partial(
        pl.kernel,
        mesh=mesh,
        out_type=jax.ShapeDtypeStruct((HIST * 8, NT0, 8, 128), jnp.float32),
        scratch_types=[
            pltpu.VMEM((NBUF, 128), jnp.int32),
            pltpu.VMEM((NBUF, 128), jnp.int32),
            pltpu.VMEM((NBUF, 128), jnp.int32),
            pltpu.VMEM((NBUF, 128, 128), jnp.float32),
            pltpu.VMEM((OBUF, 8, 8, 128), jnp.float32),
            pltpu.SemaphoreType.DMA((NBUF,)),
            pltpu.SemaphoreType.DMA((NBUF,)),
            pltpu.SemaphoreType.DMA((OBUF,)),
        ],
        compiler_params=pltpu.CompilerParams(
            use_tc_tiling_on_sc=False, needs_layout_passes=False),
    )(_body)
    return run(w2, idx3)


def kernel(tensor, weights):
    w2 = weights.reshape(IN_DIM // 2, 2 * OUT_DIM)
    idx3 = jnp.transpose(tensor).reshape(HIST, NT0, 128).astype(jnp.int32)
    y = _embedding_gather(idx3, w2)
    out = (y.reshape(HIST, 8, NT0, 8, 128)
            .transpose(2, 4, 0, 1, 3)
            .reshape(BATCH, HIST, OUT_DIM))
    return out


# speedup vs baseline: 2.7083x; 1.0910x over previous
import functools

import jax
import jax.numpy as jnp
from jax import lax
from jax.experimental import pallas as pl
from jax.experimental.pallas import tpu as pltpu
from jax.experimental.pallas import tpu_sc as plsc

IN_DIM = 1000000
OUT_DIM = 64
BATCH = 16384
HIST = 50

NC = 2
NS = 16
NW = NC * NS

NT0 = BATCH // 128
T0W = NT0 // NW
NUNIT = HIST * T0W
NBUF = 4
OBUF = 2


def _body(w2_hbm, idx_hbm, y_hbm, idxbuf, halfbuf, selbuf, gbuf, obuf,
          isem, gsem, osem):
    wid = lax.axis_index("s") * NC + lax.axis_index("c")
    t0_base = wid * T0W
    iota16 = jnp.arange(16, dtype=jnp.int32)

    def unit_hw(u):
        h = lax.div(u, T0W)
        t0 = t0_base + lax.rem(u, T0W)
        return h, t0

    def fire_idx(u, slot):
        h, t0 = unit_hw(u)
        pltpu.async_copy(idx_hbm.at[h, t0], idxbuf.at[slot], isem.at[slot])

    def wait_idx(slot):
        pltpu.make_async_copy(
            idx_hbm.at[0, 0], idxbuf.at[slot], isem.at[slot]).wait()

    def vecprep(slot):
        for s8 in range(8):
            sl = pl.ds(s8 * 16, 16)
            v = idxbuf[slot, sl]
            halfbuf[slot, sl] = lax.shift_right_logical(v, 1)
            selbuf[slot, sl] = lax.shift_left(jnp.bitwise_and(v, 1), 6)

    def fire_gather(slot):
        pltpu.async_copy(w2_hbm.at[halfbuf.at[slot]], gbuf.at[slot],
                         gsem.at[slot])

    def wait_gather(slot):
        pltpu.make_async_copy(
            w2_hbm.at[pl.ds(0, 128)], gbuf.at[slot], gsem.at[slot]).wait()

    rowvs = [iota16 + s8 * 16 for s8 in range(8)]

    def transpose(slot, oslot):
        sels = [selbuf[slot, pl.ds(s8 * 16, 16)] for s8 in range(8)]

        @pl.loop(0, 4)
        def _jb(jb):
            j0 = jb * 16

            @pl.loop(0, 16, unroll=4)
            def _k(k):
                perm = jnp.bitwise_and(iota16 + k, 15)
                jv = j0 + perm
                jdv = lax.shift_right_logical(jv, 3)
                jrv = jnp.bitwise_and(jv, 7)
                for s8 in range(8):
                    val = plsc.load_gather(
                        gbuf.at[slot], [rowvs[s8], sels[s8] + jv])
                    plsc.store_scatter(
                        obuf.at[oslot], [jdv, jrv, rowvs[s8]], val)

    def fire_out(u, oslot):
        h, t0 = unit_hw(u)
        pltpu.async_copy(obuf.at[oslot],
                         y_hbm.at[pl.ds(h * 8, 8), t0], osem.at[oslot])

    def wait_out(oslot):
        pltpu.make_async_copy(
            y_hbm.at[pl.ds(0, 8), 0], obuf.at[oslot], osem.at[oslot]).wait()

    for un in range(NBUF):
        fire_idx(un, un)
    for un in range(NBUF - 1):
        wait_idx(un)
        vecprep(un)
        fire_gather(un)

    @pl.loop(0, NUNIT, step=NBUF)
    def _outer(u0):
        for b in range(NBUF):
            u = u0 + b
            slot3 = (b + NBUF - 1) % NBUF
            oslot = b % OBUF

            @pl.when(u + NBUF < NUNIT)
            def _idx():
                fire_idx(u + NBUF, b)

            @pl.when(u + (NBUF - 1) < NUNIT)
            def _refill():
                wait_idx(slot3)
                vecprep(slot3)
                fire_gather(slot3)

            wait_gather(b)

            @pl.when(u >= OBUF)
            def _wout():
                wait_out(oslot)

            transpose(b, oslot)
            fire_out(u, oslot)

    for oslot in range(OBUF):
        wait_out(oslot)


NBLK = 7812
TAIL_COLS = IN_DIM - NBLK * 128


def _detile_body(wt_hbm, w2_hbm, ibuf0, ibuf1, obuf0, obuf1, tbuf,
                 isem, osem):
    wid = lax.axis_index("s") * NC + lax.axis_index("c")
    start = wid * 244 + 2 * jnp.minimum(wid, 2)
    cnt = jnp.where(wid < 2, 246, 244)
    iota16 = jnp.arange(16, dtype=jnp.int32)
    perms = [jnp.bitwise_and(iota16 + m, 15) for m in range(16)]
    ibufs = (ibuf0, ibuf1)
    obufs = (obuf0, obuf1)

    def fire_in(blk, slot):
        pltpu.async_copy(wt_hbm.at[pl.ds(0, 64), pl.ds(blk * 128, 128)],
                         ibufs[slot], isem.at[slot])

    def wait_in(slot):
        pltpu.make_async_copy(
            wt_hbm.at[pl.ds(0, 64), pl.ds(0, 128)], ibufs[slot],
            isem.at[slot]).wait()

    def wait_out(slot):
        pltpu.make_async_copy(
            w2_hbm.at[pl.ds(0, 8192)], obufs[slot], osem.at[slot]).wait()

    def transpose_blk(slot):
        @pl.loop(0, 32)
        def _t(t):
            cb = lax.shift_right_logical(t, 2)
            jb = jnp.bitwise_and(t, 3)
            cv = cb * 16 + iota16
            cpart = (lax.shift_left(lax.shift_right_logical(cv, 1), 7)
                     + lax.shift_left(jnp.bitwise_and(cv, 1), 6))
            j0 = jb * 16
            for m in range(16):
                jv = j0 + perms[m]
                val = plsc.load_gather(ibufs[slot], [jv, cv])
                plsc.store_scatter(obufs[slot], [cpart + jv], val)

    fire_in(start, 0)

    @pl.loop(0, cnt, step=2)
    def _outer(l0):
        for s in range(2):
            li = l0 + s
            blk = start + li

            @pl.when(li + 1 < cnt)
            def _next():
                fire_in(blk + 1, (s + 1) % 2)

            wait_in(s)

            @pl.when(li >= 2)
            def _wout():
                wait_out(s)

            transpose_blk(s)
            pltpu.async_copy(obufs[s], w2_hbm.at[pl.ds(blk * 8192, 8192)],
                             osem.at[s])

    for s in range(2):
        wait_out(s)

    @pl.when(wid == NW - 1)
    def _tail():
        pltpu.sync_copy(wt_hbm.at[pl.ds(0, 64), pl.ds(NBLK * 128, TAIL_COLS)],
                        tbuf)

        @pl.loop(0, 16)
        def _tt(t):
            cb = lax.shift_right_logical(t, 2)
            jb = jnp.bitwise_and(t, 3)
            cv = cb * 16 + iota16
            cpart = (lax.shift_left(lax.shift_right_logical(cv, 1), 7)
                     + lax.shift_left(jnp.bitwise_and(cv, 1), 6))
            j0 = jb * 16
            for m in range(16):
                jv = j0 + perms[m]
                val = plsc.load_gather(tbuf, [jv, cv])
                plsc.store_scatter(obuf0, [cpart + jv], val)
        pltpu.sync_copy(obuf0.at[pl.ds(0, 4096)],
                        w2_hbm.at[pl.ds(NBLK * 8192, 4096)])


@jax.jit
def _detile(wt):
    mesh = plsc.VectorSubcoreMesh(
        core_axis_name="c", subcore_axis_name="s",
        num_cores=NC, num_subcores=NS,
    )
    run = functools.partial(
        pl.kernel,
        mesh=mesh,
        out_type=jax.ShapeDtypeStruct((IN_DIM * OUT_DIM,), jnp.float32),
        scratch_types=[
            pltpu.VMEM((64, 128), jnp.float32),
            pltpu.VMEM((64, 128), jnp.float32),
            pltpu.VMEM((8192,), jnp.float32),
            pltpu.VMEM((8192,), jnp.float32),
            pltpu.VMEM((64, TAIL_COLS), jnp.float32),
            pltpu.SemaphoreType.DMA((2,)),
            pltpu.SemaphoreType.DMA((2,)),
        ],
        compiler_params=pltpu.CompilerParams(
            use_tc_tiling_on_sc=True, needs_layout_passes=False),
    )(_detile_body)
    return run(wt)


@jax.jit
def _embedding_gather(idx3, w2):
    mesh = plsc.VectorSubcoreMesh(
        core_axis_name="c", subcore_axis_name="s",
        num_cores=NC, num_subcores=NS,
    )
    run = functools.partial(
        pl.kernel,
        mesh=mesh,
        out_type=jax.ShapeDtypeStruct((HIST * 8, NT0, 8, 128), jnp.float32),
        scratch_types=[
            pltpu.VMEM((NBUF, 128), jnp.int32),
            pltpu.VMEM((NBUF, 128), jnp.int32),
            pltpu.VMEM((NBUF, 128), jnp.int32),
            pltpu.VMEM((NBUF, 128, 128), jnp.float32),
            pltpu.VMEM((OBUF, 8, 8, 128), jnp.float32),
            pltpu.SemaphoreType.DMA((NBUF,)),
            pltpu.SemaphoreType.DMA((NBUF,)),
            pltpu.SemaphoreType.DMA((OBUF,)),
        ],
        compiler_params=pltpu.CompilerParams(
            use_tc_tiling_on_sc=False, needs_layout_passes=False),
    )(_body)
    return run(w2, idx3)


def kernel(tensor, weights):
    w2 = _detile(jnp.transpose(weights)).reshape(IN_DIM // 2, 2 * OUT_DIM)
    idx3 = jnp.transpose(tensor).reshape(HIST, NT0, 128).astype(jnp.int32)
    y = _embedding_gather(idx3, w2)
    out = (y.reshape(HIST, 8, NT0, 8, 128)
            .transpose(2, 4, 0, 1, 3)
            .reshape(BATCH, HIST, OUT_DIM))
    return out
